# baseline (speedup 1.0000x reference)
import functools

import jax
import jax.numpy as jnp
from jax import lax
from jax.experimental import pallas as pl
from jax.experimental.pallas import tpu as pltpu
from jax.experimental.pallas import tpu_sc as plsc

_L = 16
_TILE = 128
_CHT = 8
_RING = 64
_WIN = 1024
_WL = 176


@jax.jit
def kernel(node, time, embedding):
    B = node.shape[0]
    N, D = embedding.shape
    embT = embedding.T
    n_tiles = N // _TILE + 1
    tail_n = N - (n_tiles - 1) * _TILE
    tail = jnp.pad(embT[:, N - tail_n:], ((0, 0), (0, _TILE - tail_n)))

    info = plsc.get_sparse_core_info()
    nsub = info.num_subcores
    nw = info.num_cores * nsub
    base_t, extra = divmod(n_tiles, nw)
    n_chunks = -(-(base_t + 1) // _CHT)
    stage_words = 2 * B * D

    mesh = plsc.VectorSubcoreMesh(core_axis_name="c", subcore_axis_name="s")

    @functools.partial(
        pl.kernel,
        mesh=mesh,
        compiler_params=pltpu.CompilerParams(needs_layout_passes=False),
        out_type=jax.ShapeDtypeStruct((stage_words,), jnp.float32),
        scratch_types=[
            pltpu.VMEM((2, _WIN), jnp.int32),
            pltpu.VMEM((2080,), jnp.int32),
            pltpu.VMEM((_WL,), jnp.int32),
            pltpu.VMEM((3, D, _CHT * _TILE), jnp.float32),
            pltpu.VMEM((_RING, D), jnp.float32),
            pltpu.SemaphoreType.DMA,
            pltpu.SemaphoreType.DMA,
            pltpu.SemaphoreType.DMA,
        ],
    )
    def sc_gather(node_hbm, time_hbm, embT_hbm, tail_hbm, s_hbm,
                  idxwin, hits, wl, slab, ring,
                  sem_sw, sem_st, sem_ix):
        c = lax.axis_index("c")
        s = lax.axis_index("s")
        w = c * nsub + s
        lo_t = w * base_t + jnp.minimum(w, extra)
        my_t = base_t + jnp.where(w < extra, 1, 0)
        hi_t = lo_t + my_t
        lo_n = lo_t * _TILE
        hi_n = hi_t * _TILE

        lanes = lax.iota(jnp.int32, _L)
        srcs = (node_hbm, time_hbm)
        n_pieces = B // _WIN

        last_full = n_tiles - 1

        def fire(ch):
            buf = ch - (ch // 3) * 3
            t0 = lo_t + ch * _CHT
            full_w = _CHT * _TILE

            @pl.when(t0 + _CHT <= jnp.minimum(hi_t, last_full))
            def _():
                pltpu.async_copy(
                    embT_hbm.at[:, pl.ds(
                        pl.multiple_of(t0 * _TILE, _TILE), full_w)],
                    slab.at[buf], sem_sw)

            @pl.when(t0 + _CHT > jnp.minimum(hi_t, last_full))
            def _():
                nt = jnp.clip(jnp.minimum(hi_t, last_full) - t0, 0, _CHT)

                def body(ti, carry):
                    pltpu.async_copy(
                        embT_hbm.at[:, pl.ds(
                            pl.multiple_of((t0 + ti) * _TILE, _TILE), _TILE)],
                        slab.at[buf, :, pl.ds(ti * _TILE, _TILE)], sem_sw)
                    return carry
                lax.fori_loop(0, nt, body, 0)
                @pl.when((t0 <= last_full) & (last_full < t0 + _CHT)
                         & (hi_t > last_full))
                def _():
                    pltpu.async_copy(
                        tail_hbm,
                        slab.at[buf, :, pl.ds((last_full - t0) * _TILE,
                                              _TILE)], sem_sw)

        def drain(ch):
            buf = ch - (ch // 3) * 3
            t0 = lo_t + ch * _CHT

            @pl.when(t0 + _CHT <= jnp.minimum(hi_t, last_full))
            def _():
                pltpu.make_async_copy(
                    embT_hbm.at[:, pl.ds(0, _CHT * _TILE)], slab.at[buf],
                    sem_sw).wait()

            @pl.when(t0 + _CHT > jnp.minimum(hi_t, last_full))
            def _():
                nt = jnp.clip(jnp.minimum(hi_t, last_full) - t0, 0, _CHT)
                nt = nt + jnp.where(
                    (t0 <= last_full) & (last_full < t0 + _CHT)
                    & (hi_t > last_full), 1, 0)

                def body(ti, carry):
                    pltpu.make_async_copy(
                        embT_hbm.at[:, pl.ds(0, _TILE)],
                        slab.at[buf, :, pl.ds(0, _TILE)], sem_sw).wait()
                    return carry
                lax.fori_loop(0, nt, body, 0)


        def chunk_loop(ch, m_c):
            buf = ch - (ch // 3) * 3
            t0 = lo_t + ch * _CHT

            drain(ch)

            clo = (t0 - lo_t) * _TILE << 16
            chi = (jnp.minimum(t0 + _CHT, hi_t) - lo_t) * _TILE << 16
            nv = (nh + _L - 1) >> 4

            def rescan(k, nc):
                hv = hits[pl.ds(k * _L, _L)]
                m2 = (hv >= clo) & (hv < chi)
                cnt = plsc.all_reduce_population_count(m2)[0]
                plsc.store_compressed(wl.at[pl.ds(nc, _L)], hv, mask=m2)
                return nc + cnt
            nc = lax.fori_loop(0, nv, rescan, 0)

            bufv = jnp.full((_L,), buf, jnp.int32)
            f0 = lanes
            f1 = lanes + _L
            base_n = t0 * _TILE

            def extract(e, m_e):
                pk = wl[pl.ds(e, _L)][0]
                rel = pk >> 16
                pos = pk & 0xFFFF
                lv = jnp.full((_L,), rel + lo_n - base_n, jnp.int32)
                v0 = plsc.load_gather(slab, [bufv, f0, lv])
                v1 = plsc.load_gather(slab, [bufv, f1, lv])
                slot = m_e & (_RING - 1)

                @pl.when(m_e >= _RING)
                def _():
                    pltpu.make_async_copy(
                        tail_hbm.at[0, pl.ds(0, D)], ring.at[slot],
                        sem_st).wait()
                ring[slot, pl.ds(0, _L)] = v0
                ring[slot, pl.ds(_L, _L)] = v1
                pltpu.async_copy(ring.at[slot],
                                 s_hbm.at[pl.ds(pos * D, D)], sem_st)
                return m_e + 1
            m_new = lax.fori_loop(0, nc, extract, m_c)

            @pl.when(ch + 3 < n_chunks)
            def _():
                fire(ch + 3)
            return m_new

        fire(0)
        fire(1)
        fire(2)

        _sc = jax.named_scope("idx_scan"); _sc.__enter__()
        nh = 0
        for a in range(2):
            def piece(p, nh_c, a=a):
                buf = p & 1
                pltpu.sync_copy(srcs[a].at[pl.ds(p * _WIN, _WIN)],
                                idxwin.at[buf])

                def scan(i, nh_i, a=a):
                    for u in range(4):
                        iv = idxwin[buf, pl.ds((i * 4 + u) * _L, _L)]
                        m = (iv >= lo_n) & (iv < hi_n)
                        cnt = plsc.all_reduce_population_count(m)[0]
                        pv = (a * B + p * _WIN + (i * 4 + u) * _L) + lanes
                        packed = ((iv - lo_n) << 16) | pv
                        plsc.store_compressed(
                            hits.at[pl.ds(nh_i, _L)], packed, mask=m)
                        nh_i = nh_i + cnt
                    return nh_i
                return lax.fori_loop(0, _WIN // (_L * 4), scan, nh_c)
            nh = lax.fori_loop(0, n_pieces, piece, nh)
        hits[pl.ds(nh, _L)] = jnp.full((_L,), -1, jnp.int32)
        _sc.__exit__(None, None, None)

        _sw = jax.named_scope("sweep_extract"); _sw.__enter__()
        m_fin = lax.fori_loop(0, n_chunks, chunk_loop, 0)
        _sw.__exit__(None, None, None)

        def fdrain(i, carry):
            pltpu.make_async_copy(
                tail_hbm.at[0, pl.ds(0, D)], ring.at[0], sem_st).wait()
            return carry
        lax.fori_loop(0, jnp.minimum(m_fin, _RING), fdrain, 0)

    node_i = node.astype(jnp.int32)
    time_i = time.astype(jnp.int32)
    sarr = sc_gather(node_i, time_i, embT, tail)

    rows = stage_words // _TILE
    half = rows // 2
    sr = sarr.reshape(rows, _TILE)
    blk = 1024
    grid = half // blk

    def dot_kernel(sn, st, o):
        p = sn[...] * st[...]
        seg = jax.lax.broadcasted_iota(jnp.int32, (_TILE, _TILE // D), 0) // D
        col = jax.lax.broadcasted_iota(jnp.int32, (_TILE, _TILE // D), 1)
        m = jnp.where(seg == col, 1.0, 0.0).astype(jnp.float32)
        o[...] = jax.lax.dot_general(
            p, m, (((1,), (0,)), ((), ())),
            preferred_element_type=jnp.float32)

    out4 = pl.pallas_call(
        dot_kernel,
        grid=(grid,),
        in_specs=[
            pl.BlockSpec((blk, _TILE), lambda i: (i, 0)),
            pl.BlockSpec((blk, _TILE), lambda i: (i + grid, 0)),
        ],
        out_specs=pl.BlockSpec((blk, _TILE // D), lambda i: (i, 0)),
        out_shape=jax.ShapeDtypeStruct((half, _TILE // D), jnp.float32),
    )(sr, sr)

    return out4.reshape(B)


# speedup vs baseline: 1.0502x; 1.0502x over previous
import functools

import jax
import jax.numpy as jnp
from jax import lax
from jax.experimental import pallas as pl
from jax.experimental.pallas import tpu as pltpu
from jax.experimental.pallas import tpu_sc as plsc

_L = 16
_TILE = 128
_CHT = 8
_RING = 256
_WIN = 2048
_WL = 176


@jax.jit
def kernel(node, time, embedding):
    B = node.shape[0]
    N, D = embedding.shape
    embT = embedding.T
    n_tiles = N // _TILE + 1
    tail_n = N - (n_tiles - 1) * _TILE
    tail = jnp.pad(embT[:, N - tail_n:], ((0, 0), (0, _TILE - tail_n)))

    info = plsc.get_sparse_core_info()
    nsub = info.num_subcores
    nw = info.num_cores * nsub
    base_t, extra = divmod(n_tiles, nw)
    n_chunks = -(-(base_t + 1) // _CHT)
    stage_words = 2 * B * D

    mesh = plsc.VectorSubcoreMesh(core_axis_name="c", subcore_axis_name="s")

    @functools.partial(
        pl.kernel,
        mesh=mesh,
        compiler_params=pltpu.CompilerParams(needs_layout_passes=False),
        out_type=jax.ShapeDtypeStruct((stage_words,), jnp.float32),
        scratch_types=[
            pltpu.VMEM((2, _WIN), jnp.int32),
            pltpu.VMEM((2080,), jnp.int32),
            pltpu.VMEM((_WL,), jnp.int32),
            pltpu.VMEM((2, D, _CHT * _TILE), jnp.float32),
            pltpu.VMEM((_RING, D), jnp.float32),
            pltpu.SemaphoreType.DMA,
            pltpu.SemaphoreType.DMA,
            pltpu.SemaphoreType.DMA,
        ],
    )
    def sc_gather(node_hbm, time_hbm, embT_hbm, tail_hbm, s_hbm,
                  idxwin, hits, wl, slab, ring,
                  sem_sw, sem_st, sem_ix):
        c = lax.axis_index("c")
        s = lax.axis_index("s")
        w = c * nsub + s
        lo_t = w * base_t + jnp.minimum(w, extra)
        my_t = base_t + jnp.where(w < extra, 1, 0)
        hi_t = lo_t + my_t
        lo_n = lo_t * _TILE
        hi_n = hi_t * _TILE

        lanes = lax.iota(jnp.int32, _L)
        srcs = (node_hbm, time_hbm)
        n_pieces = B // _WIN

        last_full = n_tiles - 1

        def fire(ch):
            buf = ch & 1
            t0 = lo_t + ch * _CHT
            full_w = _CHT * _TILE

            @pl.when(t0 + _CHT <= jnp.minimum(hi_t, last_full))
            def _():
                pltpu.async_copy(
                    embT_hbm.at[:, pl.ds(
                        pl.multiple_of(t0 * _TILE, _TILE), full_w)],
                    slab.at[buf], sem_sw)

            @pl.when(t0 + _CHT > jnp.minimum(hi_t, last_full))
            def _():
                nt = jnp.clip(jnp.minimum(hi_t, last_full) - t0, 0, _CHT)

                def body(ti, carry):
                    pltpu.async_copy(
                        embT_hbm.at[:, pl.ds(
                            pl.multiple_of((t0 + ti) * _TILE, _TILE), _TILE)],
                        slab.at[buf, :, pl.ds(ti * _TILE, _TILE)], sem_sw)
                    return carry
                lax.fori_loop(0, nt, body, 0)
                @pl.when((t0 <= last_full) & (last_full < t0 + _CHT)
                         & (hi_t > last_full))
                def _():
                    pltpu.async_copy(
                        tail_hbm,
                        slab.at[buf, :, pl.ds((last_full - t0) * _TILE,
                                              _TILE)], sem_sw)

        def drain(ch):
            buf = ch & 1
            t0 = lo_t + ch * _CHT

            @pl.when(t0 + _CHT <= jnp.minimum(hi_t, last_full))
            def _():
                pltpu.make_async_copy(
                    embT_hbm.at[:, pl.ds(0, _CHT * _TILE)], slab.at[buf],
                    sem_sw).wait()

            @pl.when(t0 + _CHT > jnp.minimum(hi_t, last_full))
            def _():
                nt = jnp.clip(jnp.minimum(hi_t, last_full) - t0, 0, _CHT)
                nt = nt + jnp.where(
                    (t0 <= last_full) & (last_full < t0 + _CHT)
                    & (hi_t > last_full), 1, 0)

                def body(ti, carry):
                    pltpu.make_async_copy(
                        embT_hbm.at[:, pl.ds(0, _TILE)],
                        slab.at[buf, :, pl.ds(0, _TILE)], sem_sw).wait()
                    return carry
                lax.fori_loop(0, nt, body, 0)


        def chunk_loop(ch, m_c):
            buf = ch & 1
            t0 = lo_t + ch * _CHT

            drain(ch)

            clo = (t0 - lo_t) * _TILE << 16
            chi = (jnp.minimum(t0 + _CHT, hi_t) - lo_t) * _TILE << 16
            nv = (nh + _L - 1) >> 4

            def rescan(k, nc):
                hv = hits[pl.ds(k * _L, _L)]
                m2 = (hv >= clo) & (hv < chi)
                cnt = plsc.all_reduce_population_count(m2)[0]
                plsc.store_compressed(wl.at[pl.ds(nc, _L)], hv, mask=m2)
                return nc + cnt
            nc = lax.fori_loop(0, nv, rescan, 0)

            bufv = jnp.full((_L,), buf, jnp.int32)
            f0 = lanes
            f1 = lanes + _L
            base_n = t0 * _TILE

            def extract(e, m_e):
                pk = wl[pl.ds(e, _L)][0]
                rel = pk >> 16
                pos = pk & 0xFFFF
                lv = jnp.full((_L,), rel + lo_n - base_n, jnp.int32)
                v0 = plsc.load_gather(slab, [bufv, f0, lv])
                v1 = plsc.load_gather(slab, [bufv, f1, lv])
                slot = m_e & (_RING - 1)

                @pl.when(m_e >= _RING)
                def _():
                    pltpu.make_async_copy(
                        tail_hbm.at[0, pl.ds(0, D)], ring.at[slot],
                        sem_st).wait()
                ring[slot, pl.ds(0, _L)] = v0
                ring[slot, pl.ds(_L, _L)] = v1
                pltpu.async_copy(ring.at[slot],
                                 s_hbm.at[pl.ds(pos * D, D)], sem_st)
                return m_e + 1
            m_new = lax.fori_loop(0, nc, extract, m_c)

            @pl.when(ch + 2 < n_chunks)
            def _():
                fire(ch + 2)
            return m_new

        nh = 0
        for a in range(2):
            def piece(p, nh_c, a=a):
                buf = p & 1
                pltpu.sync_copy(srcs[a].at[pl.ds(p * _WIN, _WIN)],
                                idxwin.at[buf])

                def scan(i, nh_i, a=a):
                    for u in range(4):
                        iv = idxwin[buf, pl.ds((i * 4 + u) * _L, _L)]
                        m = (iv >= lo_n) & (iv < hi_n)
                        cnt = plsc.all_reduce_population_count(m)[0]
                        pv = (a * B + p * _WIN + (i * 4 + u) * _L) + lanes
                        packed = ((iv - lo_n) << 16) | pv
                        plsc.store_compressed(
                            hits.at[pl.ds(nh_i, _L)], packed, mask=m)
                        nh_i = nh_i + cnt
                    return nh_i
                return lax.fori_loop(0, _WIN // (_L * 4), scan, nh_c)
            nh = lax.fori_loop(0, n_pieces, piece, nh)
        hits[pl.ds(nh, _L)] = jnp.full((_L,), -1, jnp.int32)

        fire(0)
        fire(1)
        m_fin = lax.fori_loop(0, n_chunks, chunk_loop, 0)

        def fdrain(i, carry):
            pltpu.make_async_copy(
                tail_hbm.at[0, pl.ds(0, D)], ring.at[0], sem_st).wait()
            return carry
        lax.fori_loop(0, jnp.minimum(m_fin, _RING), fdrain, 0)

    node_i = node.astype(jnp.int32)
    time_i = time.astype(jnp.int32)
    sarr = sc_gather(node_i, time_i, embT, tail)

    rows = stage_words // _TILE
    half = rows // 2
    sr = sarr.reshape(rows, _TILE)
    blk = 1024
    grid = half // blk

    def dot_kernel(sn, st, o):
        p = sn[...] * st[...]
        seg = jax.lax.broadcasted_iota(jnp.int32, (_TILE, _TILE // D), 0) // D
        col = jax.lax.broadcasted_iota(jnp.int32, (_TILE, _TILE // D), 1)
        m = jnp.where(seg == col, 1.0, 0.0).astype(jnp.float32)
        o[...] = jax.lax.dot_general(
            p, m, (((1,), (0,)), ((), ())),
            preferred_element_type=jnp.float32)

    out4 = pl.pallas_call(
        dot_kernel,
        grid=(grid,),
        in_specs=[
            pl.BlockSpec((blk, _TILE), lambda i: (i, 0)),
            pl.BlockSpec((blk, _TILE), lambda i: (i + grid, 0)),
        ],
        out_specs=pl.BlockSpec((blk, _TILE // D), lambda i: (i, 0)),
        out_shape=jax.ShapeDtypeStruct((half, _TILE // D), jnp.float32),
    )(sr, sr)

    return out4.reshape(B)
